# baseline (speedup 1.0000x reference)
import functools

import jax
import jax.numpy as jnp
from jax import lax
from jax.experimental import pallas as pl
from jax.experimental.pallas import tpu as pltpu
from jax.experimental.pallas import tpu_sc as plsc

MAXLEN = 8192
OUTPUT_DIM = 1024

_info = plsc.get_sparse_core_info()
NC, NS = _info.num_cores, _info.num_subcores
NW = NC * NS
ROWS_PER_W = MAXLEN // NW

_mesh = plsc.VectorSubcoreMesh(core_axis_name="c", subcore_axis_name="s")


@functools.partial(
    pl.kernel,
    mesh=_mesh,
    out_type=jax.ShapeDtypeStruct((MAXLEN, OUTPUT_DIM), jnp.float32),
    scratch_types=[pltpu.SemaphoreType.DMA],
)
def _sc_copy(table_hbm, out_hbm, sem):
    wid = lax.axis_index("s") * NC + lax.axis_index("c")
    base = wid * ROWS_PER_W
    pltpu.async_copy(
        table_hbm.at[pl.ds(base, ROWS_PER_W), :],
        out_hbm.at[pl.ds(base, ROWS_PER_W), :],
        sem,
    ).wait()


def kernel(inputs, table):
    del inputs
    return _sc_copy(table)[None]


# speedup vs baseline: 23.9541x; 23.9541x over previous
import functools

import jax
import jax.numpy as jnp
from jax import lax
from jax.experimental import pallas as pl
from jax.experimental.pallas import tpu as pltpu
from jax.experimental.pallas import tpu_sc as plsc

MAXLEN = 8192
OUTPUT_DIM = 1024

_info = plsc.get_sparse_core_info()
NC, NS = _info.num_cores, _info.num_subcores
NW = NC * NS
ROWS_PER_W = MAXLEN // NW

CHUNK = 32
NCHUNK = ROWS_PER_W // CHUNK
NBUF = 3

_mesh = plsc.VectorSubcoreMesh(core_axis_name="c", subcore_axis_name="s")


@functools.partial(
    pl.kernel,
    mesh=_mesh,
    out_type=jax.ShapeDtypeStruct((MAXLEN, OUTPUT_DIM), jnp.float32),
    scratch_types=[
        pltpu.VMEM((NBUF, CHUNK, OUTPUT_DIM), jnp.float32),
        pltpu.SemaphoreType.DMA((NBUF,)),
        pltpu.SemaphoreType.DMA((NBUF,)),
    ],
)
def _sc_copy(table_hbm, out_hbm, buf, in_sem, out_sem):
    wid = lax.axis_index("s") * NC + lax.axis_index("c")
    base = wid * ROWS_PER_W

    def in_copy(c):
        return pltpu.make_async_copy(
            table_hbm.at[pl.ds(base + c * CHUNK, CHUNK), :],
            buf.at[c % NBUF],
            in_sem.at[c % NBUF],
        )

    def out_copy(c):
        return pltpu.make_async_copy(
            buf.at[c % NBUF],
            out_hbm.at[pl.ds(base + c * CHUNK, CHUNK), :],
            out_sem.at[c % NBUF],
        )

    in_copy(0).start()
    in_copy(1).start()
    for c in range(NCHUNK):
        in_copy(c).wait()
        out_copy(c).start()
        nxt = c + 2
        if nxt < NCHUNK:
            if nxt >= NBUF:
                out_copy(nxt - NBUF).wait()
            in_copy(nxt).start()
    for c in range(NCHUNK - NBUF, NCHUNK):
        out_copy(c).wait()


def kernel(inputs, table):
    del inputs
    return _sc_copy(table)[None]


# speedup vs baseline: 48.9994x; 2.0456x over previous
import jax
import jax.numpy as jnp
from jax.experimental import pallas as pl
from jax.experimental.pallas import tpu as pltpu

MAXLEN = 8192
OUTPUT_DIM = 1024

CHUNK = 1024
NCHUNK = MAXLEN // CHUNK
NBUF = 4


def _dma_body(tab_ref, out_ref, buf, in_sem, out_sem):
    def in_copy(c):
        return pltpu.make_async_copy(
            tab_ref.at[pl.ds(c * CHUNK, CHUNK), :],
            buf.at[c % NBUF],
            in_sem.at[c % NBUF],
        )

    def out_copy(c):
        return pltpu.make_async_copy(
            buf.at[c % NBUF],
            out_ref.at[0, pl.ds(c * CHUNK, CHUNK), :],
            out_sem.at[c % NBUF],
        )

    for c in range(NBUF - 1):
        in_copy(c).start()
    for c in range(NCHUNK):
        in_copy(c).wait()
        out_copy(c).start()
        nxt = c + NBUF - 1
        if nxt < NCHUNK:
            if nxt >= NBUF:
                out_copy(nxt - NBUF).wait()
            in_copy(nxt).start()
    for c in range(NCHUNK - NBUF, NCHUNK):
        out_copy(c).wait()


def kernel(inputs, table):
    del inputs
    out = pl.pallas_call(
        _dma_body,
        in_specs=[pl.BlockSpec(memory_space=pl.ANY)],
        out_specs=pl.BlockSpec(memory_space=pl.ANY),
        out_shape=jax.ShapeDtypeStruct((1, MAXLEN, OUTPUT_DIM), table.dtype),
        scratch_shapes=[
            pltpu.VMEM((NBUF, CHUNK, OUTPUT_DIM), jnp.float32),
            pltpu.SemaphoreType.DMA((NBUF,)),
            pltpu.SemaphoreType.DMA((NBUF,)),
        ],
    )(table)
    return out
